# baseline (speedup 1.0000x reference)
import functools

import jax
import jax.numpy as jnp
from jax import lax
from jax.experimental import pallas as pl
from jax.experimental.pallas import tpu as pltpu
from jax.experimental.pallas import tpu_sc as plsc

VOCAB = 1000000
B = 16384
EMB = 64
INPUT_DIM = 2 * EMB
HID = 128
OUT = 64

NC = 2
NS = 16
NW = NC * NS
B_PER_W = B // NW
CHUNK = 128
NCHUNK = B_PER_W // CHUNK

TCOLS = 16384
BS = TCOLS.bit_length() - 1
Q = TCOLS // 4
TGRID = (VOCAB + TCOLS - 1) // TCOLS
NQUAD = TGRID * Q


def _pack_quads(x):
  u = lax.bitcast_convert_type(x, jnp.uint32)
  mask = jnp.uint32(0xFFFF0000)
  p = (u[:, 2 * Q:] & mask) | (u[:, :2 * Q] >> 16)
  pt = jnp.swapaxes(p, 0, 1)
  return lax.bitcast_convert_type(
      jnp.concatenate([pt[:Q], pt[Q:]], axis=1), jnp.int32)


def _relayout_body(a_ref, o_ref):
  o_ref[...] = _pack_quads(a_ref[...])


def _relayout(tabT):
  return pl.pallas_call(
      _relayout_body,
      grid=(TGRID,),
      in_specs=[pl.BlockSpec((EMB, TCOLS), lambda i: (0, i))],
      out_specs=pl.BlockSpec((Q, 128), lambda i: (i, 0)),
      out_shape=jax.ShapeDtypeStruct((NQUAD, 128), jnp.int32),
  )(tabT)


def _sc_gather_body(tab_hbm, idx_hbm, out_hbm, idx_v, rows_v, sem):
  wid = lax.axis_index("s") * NC + lax.axis_index("c")
  base_chunk = wid * NCHUNK
  base_row = wid * B_PER_W

  pltpu.sync_copy(idx_hbm.at[pl.ds(base_chunk, NCHUNK)], idx_v)
  copies = []
  for j in range(NCHUNK):
    copies.append(pltpu.async_copy(
        tab_hbm.at[idx_v.at[j]], rows_v.at[pl.ds(j * CHUNK, CHUNK)],
        sem))
  for c in copies:
    c.wait()
  pltpu.sync_copy(rows_v, out_hbm.at[pl.ds(base_row, B_PER_W)])


@functools.cache
def _sc_gather():
  return pl.kernel(
      _sc_gather_body,
      out_type=jax.ShapeDtypeStruct((B, 128), jnp.int32),
      mesh=plsc.VectorSubcoreMesh(core_axis_name="c", subcore_axis_name="s"),
      scratch_types=[
          pltpu.VMEM((NCHUNK, CHUNK), jnp.int32),
          pltpu.VMEM((B_PER_W, 128), jnp.int32),
          pltpu.SemaphoreType.DMA,
      ],
  )


BT = 2048


def _select_quarter(x, jlo, jhi):
  u = lax.bitcast_convert_type(x, jnp.uint32)
  hi = lax.bitcast_convert_type(u & jnp.uint32(0xFFFF0000), jnp.float32)
  lo = lax.bitcast_convert_type(u << 16, jnp.float32)
  c = jnp.where(jhi > 0.5, hi, lo)
  return jnp.where(jlo > 0.5, c[:, EMB:], c[:, :EMB])


def _mlp_body(qt_ref, ql_ref, tlo_ref, thi_ref, llo_ref, lhi_ref,
              ln1w_ref, ln1b_ref, w1t_ref, b1_ref, w2t_ref, b2_ref,
              ln2w_ref, ln2b_ref, out_ref):
  et = _select_quarter(qt_ref[...], tlo_ref[...], thi_ref[...])
  el = _select_quarter(ql_ref[...], llo_ref[...], lhi_ref[...])
  x = jnp.concatenate([et, el], axis=1)
  mu = jnp.mean(x, axis=1, keepdims=True)
  xc = x - mu
  var = jnp.mean(xc * xc, axis=1, keepdims=True)
  h = xc * jax.lax.rsqrt(var + 1e-5) * ln1w_ref[...] + ln1b_ref[...]
  h = jnp.dot(h, w1t_ref[...], preferred_element_type=jnp.float32)
  h = jnp.maximum(h + b1_ref[...], 0.0)
  y = jnp.dot(h, w2t_ref[...], preferred_element_type=jnp.float32)
  y = y + b2_ref[...]
  mu2 = jnp.mean(y, axis=1, keepdims=True)
  yc = y - mu2
  var2 = jnp.mean(yc * yc, axis=1, keepdims=True)
  out_ref[...] = yc * jax.lax.rsqrt(var2 + 1e-5) * ln2w_ref[...] + ln2b_ref[...]


def _mlp(qt, ql, tlo, thi, llo, lhi, ln1w, ln1b, w1t, b1, w2t, b2,
         ln2w, ln2b):
  full = lambda shape: pl.BlockSpec(shape, lambda i: tuple(0 for _ in shape))
  par = pl.BlockSpec((BT, 1), lambda i: (i, 0))
  return pl.pallas_call(
      _mlp_body,
      grid=(B // BT,),
      in_specs=[
          pl.BlockSpec((BT, 128), lambda i: (i, 0)),
          pl.BlockSpec((BT, 128), lambda i: (i, 0)),
          par, par, par, par,
          full((1, INPUT_DIM)), full((1, INPUT_DIM)),
          full((INPUT_DIM, HID)), full((1, HID)),
          full((HID, OUT)), full((1, OUT)),
          full((1, OUT)), full((1, OUT)),
      ],
      out_specs=pl.BlockSpec((BT, OUT), lambda i: (i, 0)),
      out_shape=jax.ShapeDtypeStruct((B, OUT), jnp.float32),
  )(qt, ql, tlo, thi, llo, lhi, ln1w, ln1b, w1t, b1, w2t, b2, ln2w, ln2b)


def kernel(Type, Location, emb_type, emb_loc, ln1_w, ln1_b, w1, b1, w2, b2,
           ln2_w, ln2_b):
  it = Type.astype(jnp.int32)
  il = Location.astype(jnp.int32)
  quad = lambda r: ((r >> BS) << (BS - 2)) | (r & (Q - 1))
  jlo = lambda r: ((r >> (BS - 2)) & 1).astype(jnp.float32).reshape(B, 1)
  jhi = lambda r: ((r >> (BS - 1)) & 1).astype(jnp.float32).reshape(B, 1)
  idx_t = jnp.reshape(quad(it), (B // CHUNK, CHUNK))
  idx_l = jnp.reshape(quad(il), (B // CHUNK, CHUNK))
  qtab_t = _relayout(emb_type.T)
  qt = _sc_gather()(qtab_t, idx_t)
  qtab_l = _relayout(emb_loc.T)
  ql = _sc_gather()(qtab_l, idx_l)
  return _mlp(
      qt, ql, jlo(it), jhi(it), jlo(il), jhi(il),
      ln1_w.reshape(1, INPUT_DIM), ln1_b.reshape(1, INPUT_DIM),
      w1.T, b1.reshape(1, HID),
      w2.T, b2.reshape(1, OUT),
      ln2_w.reshape(1, OUT), ln2_b.reshape(1, OUT),
  )


# speedup vs baseline: 1.1344x; 1.1344x over previous
import functools

import jax
import jax.numpy as jnp
from jax import lax
from jax.experimental import pallas as pl
from jax.experimental.pallas import tpu as pltpu
from jax.experimental.pallas import tpu_sc as plsc

VOCAB = 1000000
B = 16384
EMB = 64
INPUT_DIM = 2 * EMB
HID = 128
OUT = 64

NC = 2
NS = 16
NW = NC * NS
B_PER_W = B // NW
CHUNK = 128
NCHUNK = B_PER_W // CHUNK

TCOLS = 16384
BS = TCOLS.bit_length() - 1
Q = TCOLS // 4
TGRID = (VOCAB + TCOLS - 1) // TCOLS
NQUAD = TGRID * Q


def _pack_quads(x):
  u = lax.bitcast_convert_type(x, jnp.uint32)
  mask = jnp.uint32(0xFFFF0000)
  p = (u[:, 2 * Q:] & mask) | (u[:, :2 * Q] >> 16)
  pt = jnp.swapaxes(p, 0, 1)
  return lax.bitcast_convert_type(
      jnp.concatenate([pt[:Q], pt[Q:]], axis=1), jnp.int32)


def _relayout_body(at_ref, al_ref, ot_ref, ol_ref):
  ot_ref[...] = _pack_quads(at_ref[...])
  ol_ref[...] = _pack_quads(al_ref[...])


def _relayout(tabT_t, tabT_l):
  return pl.pallas_call(
      _relayout_body,
      grid=(TGRID,),
      in_specs=[
          pl.BlockSpec((EMB, TCOLS), lambda i: (0, i)),
          pl.BlockSpec((EMB, TCOLS), lambda i: (0, i)),
      ],
      out_specs=[
          pl.BlockSpec((Q, 128), lambda i: (i, 0)),
          pl.BlockSpec((Q, 128), lambda i: (i, 0)),
      ],
      out_shape=[
          jax.ShapeDtypeStruct((NQUAD, 128), jnp.int32),
          jax.ShapeDtypeStruct((NQUAD, 128), jnp.int32),
      ],
  )(tabT_t, tabT_l)


def _sc_gather_body(tab_t_hbm, tab_l_hbm, idx_t_hbm, idx_l_hbm,
                    out_t_hbm, out_l_hbm, idx_v, rows_v, sem):
  wid = lax.axis_index("s") * NC + lax.axis_index("c")
  base_chunk = wid * NCHUNK
  base_row = wid * B_PER_W

  for tab_hbm, idx_hbm, out_hbm in (
      (tab_t_hbm, idx_t_hbm, out_t_hbm),
      (tab_l_hbm, idx_l_hbm, out_l_hbm),
  ):
    pltpu.sync_copy(idx_hbm.at[pl.ds(base_chunk, NCHUNK)], idx_v)
    copies = []
    for j in range(NCHUNK):
      copies.append(pltpu.async_copy(
          tab_hbm.at[idx_v.at[j]], rows_v.at[pl.ds(j * CHUNK, CHUNK)],
          sem))
    for c in copies:
      c.wait()
    pltpu.sync_copy(rows_v, out_hbm.at[pl.ds(base_row, B_PER_W)])


@functools.cache
def _sc_gather():
  return pl.kernel(
      _sc_gather_body,
      out_type=(
          jax.ShapeDtypeStruct((B, 128), jnp.int32),
          jax.ShapeDtypeStruct((B, 128), jnp.int32),
      ),
      mesh=plsc.VectorSubcoreMesh(core_axis_name="c", subcore_axis_name="s"),
      scratch_types=[
          pltpu.VMEM((NCHUNK, CHUNK), jnp.int32),
          pltpu.VMEM((B_PER_W, 128), jnp.int32),
          pltpu.SemaphoreType.DMA,
      ],
  )


BT = 2048


def _select_quarter(x, jlo, jhi):
  u = lax.bitcast_convert_type(x, jnp.uint32)
  hi = lax.bitcast_convert_type(u & jnp.uint32(0xFFFF0000), jnp.float32)
  lo = lax.bitcast_convert_type(u << 16, jnp.float32)
  c = jnp.where(jhi > 0.5, hi, lo)
  return jnp.where(jlo > 0.5, c[:, EMB:], c[:, :EMB])


def _mlp_body(qt_ref, ql_ref, tlo_ref, thi_ref, llo_ref, lhi_ref,
              ln1w_ref, ln1b_ref, w1t_ref, b1_ref, w2t_ref, b2_ref,
              ln2w_ref, ln2b_ref, out_ref):
  et = _select_quarter(qt_ref[...], tlo_ref[...], thi_ref[...])
  el = _select_quarter(ql_ref[...], llo_ref[...], lhi_ref[...])
  x = jnp.concatenate([et, el], axis=1)
  mu = jnp.mean(x, axis=1, keepdims=True)
  xc = x - mu
  var = jnp.mean(xc * xc, axis=1, keepdims=True)
  h = xc * jax.lax.rsqrt(var + 1e-5) * ln1w_ref[...] + ln1b_ref[...]
  h = jnp.dot(h, w1t_ref[...], preferred_element_type=jnp.float32)
  h = jnp.maximum(h + b1_ref[...], 0.0)
  y = jnp.dot(h, w2t_ref[...], preferred_element_type=jnp.float32)
  y = y + b2_ref[...]
  mu2 = jnp.mean(y, axis=1, keepdims=True)
  yc = y - mu2
  var2 = jnp.mean(yc * yc, axis=1, keepdims=True)
  out_ref[...] = yc * jax.lax.rsqrt(var2 + 1e-5) * ln2w_ref[...] + ln2b_ref[...]


def _mlp(qt, ql, tlo, thi, llo, lhi, ln1w, ln1b, w1t, b1, w2t, b2,
         ln2w, ln2b):
  full = lambda shape: pl.BlockSpec(shape, lambda i: tuple(0 for _ in shape))
  par = pl.BlockSpec((BT, 1), lambda i: (i, 0))
  return pl.pallas_call(
      _mlp_body,
      grid=(B // BT,),
      in_specs=[
          pl.BlockSpec((BT, 128), lambda i: (i, 0)),
          pl.BlockSpec((BT, 128), lambda i: (i, 0)),
          par, par, par, par,
          full((1, INPUT_DIM)), full((1, INPUT_DIM)),
          full((INPUT_DIM, HID)), full((1, HID)),
          full((HID, OUT)), full((1, OUT)),
          full((1, OUT)), full((1, OUT)),
      ],
      out_specs=pl.BlockSpec((BT, OUT), lambda i: (i, 0)),
      out_shape=jax.ShapeDtypeStruct((B, OUT), jnp.float32),
  )(qt, ql, tlo, thi, llo, lhi, ln1w, ln1b, w1t, b1, w2t, b2, ln2w, ln2b)


def kernel(Type, Location, emb_type, emb_loc, ln1_w, ln1_b, w1, b1, w2, b2,
           ln2_w, ln2_b):
  it = Type.astype(jnp.int32)
  il = Location.astype(jnp.int32)
  quad = lambda r: ((r >> BS) << (BS - 2)) | (r & (Q - 1))
  jlo = lambda r: ((r >> (BS - 2)) & 1).astype(jnp.float32).reshape(B, 1)
  jhi = lambda r: ((r >> (BS - 1)) & 1).astype(jnp.float32).reshape(B, 1)
  idx_t = jnp.reshape(quad(it), (B // CHUNK, CHUNK))
  idx_l = jnp.reshape(quad(il), (B // CHUNK, CHUNK))
  qtab_t, qtab_l = _relayout(emb_type.T, emb_loc.T)
  qt, ql = _sc_gather()(qtab_t, qtab_l, idx_t, idx_l)
  return _mlp(
      qt, ql, jlo(it), jhi(it), jlo(il), jhi(il),
      ln1_w.reshape(1, INPUT_DIM), ln1_b.reshape(1, INPUT_DIM),
      w1.T, b1.reshape(1, HID),
      w2.T, b2.reshape(1, OUT),
      ln2_w.reshape(1, OUT), ln2_b.reshape(1, OUT),
  )
